# baseline (speedup 1.0000x reference)
import functools

import jax
import jax.numpy as jnp
from jax import lax
from jax.experimental import pallas as pl
from jax.experimental.pallas import tpu as pltpu
from jax.experimental.pallas import tpu_sc as plsc

_BETA = 15.0
_BATCH = 16384
_D = 16
_NW = 32
_BPW = _BATCH // _NW
_GROUPS = _BPW // 16


def _sc_kernel(user_hbm, item_hbm, ideal_hbm, mid_hbm, spread_hbm, out_hbm,
               uidx_v, iidx_v, x_v, m_v, s_v, out_v, sem):
    wid = lax.axis_index("s") * 2 + lax.axis_index("c")
    base = wid * _BPW

    pltpu.sync_copy(user_hbm.at[pl.ds(base, _BPW)], uidx_v)
    pltpu.sync_copy(item_hbm.at[pl.ds(base, _BPW)], iidx_v)

    cp_x = pltpu.async_copy(ideal_hbm.at[uidx_v], x_v, sem)
    cp_m = pltpu.async_copy(mid_hbm.at[iidx_v], m_v, sem)
    cp_s = pltpu.async_copy(spread_hbm.at[iidx_v], s_v, sem)
    cp_x.wait()
    cp_m.wait()
    cp_s.wait()

    lane = lax.iota(jnp.int32, 16)

    def body(g, _):
        rows = g * 16 + lane
        acc = jnp.zeros((16,), jnp.float32)
        for d in range(_D):
            cols = jnp.full((16,), d, jnp.int32)
            xv = plsc.load_gather(x_v, [rows, cols])
            mv = plsc.load_gather(m_v, [rows, cols])
            sv = plsc.load_gather(s_v, [rows, cols])
            acc = acc + (xv - mv) * sv
        out_v[pl.ds(g * 16, 16)] = acc * _BETA
        return _

    lax.fori_loop(0, _GROUPS, body, None)

    pltpu.sync_copy(out_v, out_hbm.at[pl.ds(base, _BPW)])


@jax.jit
def kernel(user_idx, item_idx, ideal_points, vote_midpoints, vote_spreads):
    mesh = plsc.VectorSubcoreMesh(core_axis_name="c", subcore_axis_name="s")
    run = functools.partial(
        pl.kernel,
        mesh=mesh,
        out_type=jax.ShapeDtypeStruct((_BATCH,), jnp.float32),
        scratch_types=[
            pltpu.VMEM((_BPW,), jnp.int32),
            pltpu.VMEM((_BPW,), jnp.int32),
            pltpu.VMEM((_BPW, _D), jnp.float32),
            pltpu.VMEM((_BPW, _D), jnp.float32),
            pltpu.VMEM((_BPW, _D), jnp.float32),
            pltpu.VMEM((_BPW,), jnp.float32),
            pltpu.SemaphoreType.DMA,
        ],
        compiler_params=pltpu.CompilerParams(
            needs_layout_passes=False, use_tc_tiling_on_sc=False),
    )(_sc_kernel)
    return run(user_idx.astype(jnp.int32), item_idx.astype(jnp.int32),
               ideal_points, vote_midpoints, vote_spreads)
